# baseline (speedup 1.0000x reference)
import jax
import jax.numpy as jnp
from jax import lax
from jax.experimental import pallas as pl
from jax.experimental.pallas import tpu as pltpu, tpu_sc as plsc

N = 10000
E = 160000
IN_DIM = 128
DIM = 64
HEADS = 4
NLAYERS = 3

NC = 2
NS = 16
NW = NC * NS
EPW = E // NW
C = 200
NCHUNK = EPW // C
RPT = 624
REM = N - RPT * NS

f32 = jnp.float32


def _proj_in_body(x_ref, w_ref, b_ref, o_ref):
    o_ref[...] = jax.nn.relu(
        jnp.dot(x_ref[...], w_ref[...].T, preferred_element_type=f32) + b_ref[...])


def _proj_out_body(h_ref, w_ref, b_ref, o_ref):
    o_ref[...] = jnp.dot(h_ref[...], w_ref[...].T, preferred_element_type=f32) + b_ref[...]


def _pre_body(h_ref, k_ref, wq_ref, wv_ref, wkw_ref, wkb_ref, a_ref, gw_ref,
              gb_ref, v0_ref, v1_ref, v2_ref, v3_ref, st_ref, sm_ref, g_ref):
    h = h_ref[...]
    kcol = k_ref[...]
    nb = h.shape[0]
    ku = kcol * wkw_ref[...] + wkb_ref[...]
    q = jnp.dot(h, wq_ref[...].T, preferred_element_type=f32)
    v = jnp.dot(h, wv_ref[...].T, preferred_element_type=f32)
    a = a_ref[...]
    a_k = a[:, :DIM]
    sd = jnp.dot(ku, a_k.T, preferred_element_type=f32)
    sd_parts = []
    ss_parts = []
    for hh in range(HEADS):
        a_q = a[hh:hh + 1, DIM:2 * DIM]
        a_v = a[hh:hh + 1, 2 * DIM:]
        qh = q[:, hh * DIM:(hh + 1) * DIM]
        vh = v[:, hh * DIM:(hh + 1) * DIM]
        sd_parts.append(jnp.sum(qh * a_q, axis=1, keepdims=True))
        ss_parts.append(jnp.sum(vh * a_v, axis=1, keepdims=True))
    sd = sd + jnp.concatenate(sd_parts, axis=1)
    ss = jnp.concatenate(ss_parts, axis=1)
    ssmax = jnp.max(ss, axis=0, keepdims=True)
    st_ref[...] = jnp.concatenate([sd, ss], axis=1)
    cur = jnp.concatenate([ssmax] * 4, axis=1)
    step = pl.program_id(0)

    @pl.when(step == 0)
    def _init_sm():
        sm_ref[...] = cur

    @pl.when(step != 0)
    def _acc_sm():
        sm_ref[...] = jnp.maximum(sm_ref[...], cur)
    v0_ref[...] = v[:, :DIM]
    v1_ref[...] = v[:, DIM:2 * DIM]
    v2_ref[...] = v[:, 2 * DIM:3 * DIM]
    v3_ref[...] = v[:, 3 * DIM:]
    g_ref[...] = jax.nn.sigmoid(
        jnp.dot(ku, gw_ref[...].T, preferred_element_type=f32) + gb_ref[...])


def _post_body(h_ref, agg_ref, s_ref, g_ref, wm_ref, b_ref, lng_ref, lnb_ref,
               o_ref):
    h = h_ref[...]
    s = s_ref[0, :, :HEADS] + s_ref[1, :, :HEADS] + 1e-16
    wm = wm_ref[...]
    out = h
    for hh in range(HEADS):
        agg_h = agg_ref[0, hh] + agg_ref[1, hh]
        agg_h = agg_h / jnp.broadcast_to(s[:, hh:hh + 1], agg_h.shape)
        out = out + jnp.dot(agg_h, wm[:, hh * DIM:(hh + 1) * DIM].T,
                            preferred_element_type=f32)
    mu = jnp.mean(out, axis=-1, keepdims=True)
    var = jnp.mean((out - mu) ** 2, axis=-1, keepdims=True)
    out = (out - mu) * jax.lax.rsqrt(var + 1e-5) * lng_ref[...] + lnb_ref[...]
    out = jax.nn.relu(out * (1.0 + g_ref[...]) + b_ref[...])
    o_ref[...] = out


def _tc_call(body, out_shapes, *args):
    return pl.pallas_call(body, out_shape=out_shapes)(*args)


NB = 2000


def _row_spec(cols):
    return pl.BlockSpec((NB, cols), lambda i: (i, 0))


def _fix_spec(shape):
    nd = len(shape)
    return pl.BlockSpec(shape, lambda i, _n=nd: (0,) * _n)


def _pre_call(h, kcol, wq, wv, wkw, wkb, a, gw, gb):
    return pl.pallas_call(
        _pre_body,
        grid=(N // NB,),
        in_specs=[
            _row_spec(DIM), _row_spec(1),
            _fix_spec((HEADS * DIM, DIM)), _fix_spec((HEADS * DIM, DIM)),
            _fix_spec((1, DIM)), _fix_spec((1, DIM)),
            _fix_spec((HEADS, 3 * DIM)),
            _fix_spec((DIM, DIM)), _fix_spec((1, DIM)),
        ],
        out_specs=[
            _row_spec(DIM), _row_spec(DIM), _row_spec(DIM), _row_spec(DIM),
            _row_spec(8), _fix_spec((1, 16)), _row_spec(DIM),
        ],
        out_shape=[
            jax.ShapeDtypeStruct((N, DIM), f32),
            jax.ShapeDtypeStruct((N, DIM), f32),
            jax.ShapeDtypeStruct((N, DIM), f32),
            jax.ShapeDtypeStruct((N, DIM), f32),
            jax.ShapeDtypeStruct((N, 8), f32),
            jax.ShapeDtypeStruct((1, 16), f32),
            jax.ShapeDtypeStruct((N, DIM), f32),
        ],
    )(h, kcol, wq, wv, wkw, wkb, a, gw, gb)


def _post_call(h, agg, s, g, wm, b, lng, lnb):
    return pl.pallas_call(
        _post_body,
        grid=(N // NB,),
        in_specs=[
            _row_spec(DIM),
            pl.BlockSpec((NC, HEADS, NB, DIM), lambda i: (0, 0, i, 0)),
            pl.BlockSpec((NC, NB, 8), lambda i: (0, i, 0)),
            _row_spec(DIM),
            _fix_spec((DIM, HEADS * DIM)),
            _fix_spec((1, DIM)), _fix_spec((1, DIM)), _fix_spec((1, DIM)),
        ],
        out_specs=_row_spec(DIM),
        out_shape=jax.ShapeDtypeStruct((N, DIM), f32),
    )(h, agg, s, g, wm, b, lng, lnb)


def _pre_compute(h, kcol, wq, wv, wkw, wkb, a, gw, gb,
                 v0_ref, v1_ref, v2_ref, v3_ref, st_ref, sm_ref, g_ref):
    ku = kcol * wkw + wkb
    q = jnp.dot(h, wq.T, preferred_element_type=f32)
    v = jnp.dot(h, wv.T, preferred_element_type=f32)
    a_k = a[:, :DIM]
    sd = jnp.dot(ku, a_k.T, preferred_element_type=f32)
    sd_parts = []
    ss_parts = []
    for hh in range(HEADS):
        a_q = a[hh:hh + 1, DIM:2 * DIM]
        a_v = a[hh:hh + 1, 2 * DIM:]
        qh = q[:, hh * DIM:(hh + 1) * DIM]
        vh = v[:, hh * DIM:(hh + 1) * DIM]
        sd_parts.append(jnp.sum(qh * a_q, axis=1, keepdims=True))
        ss_parts.append(jnp.sum(vh * a_v, axis=1, keepdims=True))
    sd = sd + jnp.concatenate(sd_parts, axis=1)
    ss = jnp.concatenate(ss_parts, axis=1)
    ssmax = jnp.max(ss, axis=0, keepdims=True)
    st_ref[...] = jnp.concatenate([sd, ss], axis=1)
    cur = jnp.concatenate([ssmax] * 4, axis=1)
    step = pl.program_id(0)

    @pl.when(step == 0)
    def _init_sm():
        sm_ref[...] = cur

    @pl.when(step != 0)
    def _acc_sm():
        sm_ref[...] = jnp.maximum(sm_ref[...], cur)
    v0_ref[...] = v[:, :DIM]
    v1_ref[...] = v[:, DIM:2 * DIM]
    v2_ref[...] = v[:, 2 * DIM:3 * DIM]
    v3_ref[...] = v[:, 3 * DIM:]
    g_ref[...] = jax.nn.sigmoid(
        jnp.dot(ku, gw.T, preferred_element_type=f32) + gb)


def _post_compute(h, agg_ref, s_ref, g, wm, b, lng, lnb):
    s = s_ref[0, :, :HEADS] + s_ref[1, :, :HEADS] + 1e-16
    out = h
    for hh in range(HEADS):
        agg_h = agg_ref[0, hh] + agg_ref[1, hh]
        agg_h = agg_h / jnp.broadcast_to(s[:, hh:hh + 1], agg_h.shape)
        out = out + jnp.dot(agg_h, wm[:, hh * DIM:(hh + 1) * DIM].T,
                            preferred_element_type=f32)
    mu = jnp.mean(out, axis=-1, keepdims=True)
    var = jnp.mean((out - mu) ** 2, axis=-1, keepdims=True)
    out = (out - mu) * jax.lax.rsqrt(var + 1e-5) * lng + lnb
    return jax.nn.relu(out * (1.0 + g) + b)


def _fuse0_body(x_ref, k_ref, winw_ref, winb_ref, wq_ref, wv_ref, wkw_ref,
                wkb_ref, a_ref, gw_ref, gb_ref,
                h_ref, v0_ref, v1_ref, v2_ref, v3_ref, st_ref, sm_ref, g_ref):
    h = jax.nn.relu(jnp.dot(x_ref[...], winw_ref[...].T,
                            preferred_element_type=f32) + winb_ref[...])
    h_ref[...] = h
    _pre_compute(h, k_ref[...], wq_ref[...], wv_ref[...], wkw_ref[...],
                 wkb_ref[...], a_ref[...], gw_ref[...], gb_ref[...],
                 v0_ref, v1_ref, v2_ref, v3_ref, st_ref, sm_ref, g_ref)


def _fusemid_body(h_ref, agg_ref, s_ref, gin_ref, wm_ref, b_ref, lng_ref,
                  lnb_ref, k_ref, wq_ref, wv_ref, wkw_ref, wkb_ref, a_ref,
                  gw_ref, gb_ref,
                  h_out, v0_ref, v1_ref, v2_ref, v3_ref, st_ref, sm_ref, g_ref):
    h2 = _post_compute(h_ref[...], agg_ref, s_ref, gin_ref[...], wm_ref[...],
                       b_ref[...], lng_ref[...], lnb_ref[...])
    h_out[...] = h2
    _pre_compute(h2, k_ref[...], wq_ref[...], wv_ref[...], wkw_ref[...],
                 wkb_ref[...], a_ref[...], gw_ref[...], gb_ref[...],
                 v0_ref, v1_ref, v2_ref, v3_ref, st_ref, sm_ref, g_ref)


def _fuselast_body(h_ref, agg_ref, s_ref, gin_ref, wm_ref, b_ref, lng_ref,
                   lnb_ref, woutw_ref, woutb_ref, o_ref):
    h2 = _post_compute(h_ref[...], agg_ref, s_ref, gin_ref[...], wm_ref[...],
                       b_ref[...], lng_ref[...], lnb_ref[...])
    o_ref[...] = jnp.dot(h2, woutw_ref[...].T,
                         preferred_element_type=f32) + woutb_ref[...]


_PRE_W_SPECS = [
    _fix_spec((HEADS * DIM, DIM)), _fix_spec((HEADS * DIM, DIM)),
    _fix_spec((1, DIM)), _fix_spec((1, DIM)), _fix_spec((HEADS, 3 * DIM)),
    _fix_spec((DIM, DIM)), _fix_spec((1, DIM)),
]
_PRE_OUT_SPECS = [
    _row_spec(DIM), _row_spec(DIM), _row_spec(DIM), _row_spec(DIM),
    _row_spec(8), _fix_spec((1, 16)), _row_spec(DIM),
]
_PRE_OUT_SHAPES = [
    jax.ShapeDtypeStruct((N, DIM), f32),
    jax.ShapeDtypeStruct((N, DIM), f32),
    jax.ShapeDtypeStruct((N, DIM), f32),
    jax.ShapeDtypeStruct((N, DIM), f32),
    jax.ShapeDtypeStruct((N, 8), f32),
    jax.ShapeDtypeStruct((1, 16), f32),
    jax.ShapeDtypeStruct((N, DIM), f32),
]
_POST_IN_SPECS = [
    _row_spec(DIM),
    pl.BlockSpec((NC, HEADS, NB, DIM), lambda i: (0, 0, i, 0)),
    pl.BlockSpec((NC, NB, 8), lambda i: (0, i, 0)),
    _row_spec(DIM),
    _fix_spec((DIM, HEADS * DIM)),
    _fix_spec((1, DIM)), _fix_spec((1, DIM)), _fix_spec((1, DIM)),
]


def _fuse0_call(x, kcol, winw, winb, wq, wv, wkw, wkb, a, gw, gb):
    return pl.pallas_call(
        _fuse0_body,
        grid=(N // NB,),
        in_specs=[_row_spec(IN_DIM), _row_spec(1),
                  _fix_spec((DIM, IN_DIM)), _fix_spec((1, DIM))] + _PRE_W_SPECS,
        out_specs=[_row_spec(DIM)] + _PRE_OUT_SPECS,
        out_shape=[jax.ShapeDtypeStruct((N, DIM), f32)] + _PRE_OUT_SHAPES,
    )(x, kcol, winw, winb, wq, wv, wkw, wkb, a, gw, gb)


def _fusemid_call(h, agg, s, g, wm, b, lng, lnb, kcol, wq, wv, wkw, wkb, a,
                  gw, gb):
    return pl.pallas_call(
        _fusemid_body,
        grid=(N // NB,),
        in_specs=_POST_IN_SPECS + [_row_spec(1)] + _PRE_W_SPECS,
        out_specs=[_row_spec(DIM)] + _PRE_OUT_SPECS,
        out_shape=[jax.ShapeDtypeStruct((N, DIM), f32)] + _PRE_OUT_SHAPES,
    )(h, agg, s, g, wm, b, lng, lnb, kcol, wq, wv, wkw, wkb, a, gw, gb)


def _fuselast_call(h, agg, s, g, wm, b, lng, lnb, woutw, woutb):
    return pl.pallas_call(
        _fuselast_body,
        grid=(N // NB,),
        in_specs=_POST_IN_SPECS + [_fix_spec((DIM, DIM)), _fix_spec((1, DIM))],
        out_specs=_row_spec(DIM),
        out_shape=jax.ShapeDtypeStruct((N, DIM), f32),
    )(h, agg, s, g, wm, b, lng, lnb, woutw, woutb)


def _edge_kernel(dst_hbm, src_hbm, st_hbm, smax_hbm, v0, v1, v2, v3,
                 agg_out, s_out,
                 dst2, src2, drows_a, srows_a, vbuf_a, drows_b, srows_b,
                 vbuf_b, w4a, w4b, w4all, smaxv, agg_sh, s_sh,
                 gsem_a, gsem_b, ssem_a, ssem_b):
    cid = lax.axis_index("c")
    sid = lax.axis_index("s")
    wid = sid * NC + cid

    iota = lax.iota(jnp.int32, 16)
    lane4 = iota % 4
    edge4 = iota // 4
    zeros16 = jnp.zeros((16,), f32)

    def _full(val):
        return jnp.full((16,), val, jnp.int32)

    pltpu.sync_copy(smax_hbm, smaxv)
    smax16 = smaxv[...]

    pltpu.sync_copy(dst_hbm.at[wid], dst2)
    pltpu.sync_copy(src_hbm.at[wid], src2)

    def _zv(i, _):
        plsc.store_scatter(vbuf_a, [_full(i // 4), (i % 4) * 16 + iota], zeros16)
        return _

    def _z3(i, _):
        p = i * 16 + iota
        plsc.store_scatter(w4a, [p // 8, p % 8], zeros16)
        plsc.store_scatter(w4b, [p // 8, p % 8], zeros16)
        return _
    lax.fori_loop(0, C // 2, _z3, None)

    for hh in range(HEADS):
        v_hbm = (v0, v1, v2, v3)[hh]
        lax.fori_loop(0, C * DIM // 16, _zv, None)
        for zc in range(3):
            pltpu.sync_copy(vbuf_a, agg_sh.at[pl.ds(sid * RPT + zc * C, C)])
        pltpu.sync_copy(vbuf_a.at[pl.ds(0, 24)],
                        agg_sh.at[pl.ds(sid * RPT + 3 * C, 24)])
        if hh == 0:
            for zc in range(3):
                pltpu.sync_copy(w4a, s_sh.at[pl.ds(sid * RPT + zc * C, C)])
            pltpu.sync_copy(w4a.at[pl.ds(0, 24)],
                            s_sh.at[pl.ds(sid * RPT + 3 * C, 24)])

        @pl.when(sid == 0)
        def _zero_rem():
            pltpu.sync_copy(vbuf_a.at[pl.ds(0, REM)],
                            agg_sh.at[pl.ds(NS * RPT, REM)])
            if hh == 0:
                pltpu.sync_copy(w4a.at[pl.ds(0, REM)],
                                s_sh.at[pl.ds(NS * RPT, REM)])

        plsc.subcore_barrier()

        A = (drows_a, srows_a, vbuf_a, w4a, gsem_a, ssem_a)
        B = (drows_b, srows_b, vbuf_b, w4b, gsem_b, ssem_b)

        def gstart(kk, S):
            dr, sr, vb, _, gsem, _ = S
            if hh == 0:
                pltpu.async_copy(st_hbm.at[dst2.at[kk]], dr, gsem)
                pltpu.async_copy(st_hbm.at[src2.at[kk]], sr, gsem)
            pltpu.async_copy(v_hbm.at[src2.at[kk]], vb, gsem)

        def gwait(kk, S):
            dr, sr, vb, _, gsem, _ = S
            if hh == 0:
                pltpu.make_async_copy(st_hbm.at[dst2.at[kk]], dr, gsem).wait()
                pltpu.make_async_copy(st_hbm.at[src2.at[kk]], sr, gsem).wait()
            pltpu.make_async_copy(v_hbm.at[src2.at[kk]], vb, gsem).wait()

        def compute(kk, S):
            dr, sr, vb, w4, gsem, ssem = S

            if hh == 0:
                def wstep(g, _):
                    r = g * 4 + edge4
                    sdv = plsc.load_gather(dr, [r, lane4])
                    ssv = plsc.load_gather(sr, [r, lane4 + 4])
                    tb = sdv + smax16
                    ubv = jnp.maximum(tb, 0.01 * tb)
                    t = sdv + ssv
                    t = jnp.maximum(t, 0.01 * t)
                    w = jnp.exp(t - ubv)
                    plsc.store_scatter(w4, [r, lane4], w)
                    plsc.store_scatter(w4all, [kk * C + r, lane4], w)
                    return _
                lax.fori_loop(0, C // 4, wstep, None, unroll=4)

            def mstep(e, _):
                wv = plsc.load_gather(w4all, [_full(kk * C + e), _full(hh)])
                for j in range(DIM // 16):
                    xv = vb[e, pl.ds(j * 16, 16)]
                    vb[e, pl.ds(j * 16, 16)] = xv * wv
                return _
            lax.fori_loop(0, C, mstep, None, unroll=8)

            pltpu.async_copy(vb, agg_sh.at[dst2.at[kk]], ssem, add=True)
            if hh == 0:
                pltpu.async_copy(w4, s_sh.at[dst2.at[kk]], ssem, add=True)

        def swait(kk, S):
            _, _, vb, w4, _, ssem = S
            pltpu.make_async_copy(vb, agg_sh.at[dst2.at[kk]], ssem).wait()
            if hh == 0:
                pltpu.make_async_copy(w4, s_sh.at[dst2.at[kk]], ssem).wait()

        gstart(0, A)

        def pair(kp, _):
            kk = 2 * kp
            gwait(kk, A)

            @pl.when(kp > 0)
            def _wb():
                swait(kk - 1, B)

            gstart(kk + 1, B)
            compute(kk, A)
            gwait(kk + 1, B)
            swait(kk, A)
            gstart(kk + 2, A)
            compute(kk + 1, B)
            return _
        lax.fori_loop(0, (NCHUNK - 1) // 2, pair, None)

        gwait(NCHUNK - 1, A)
        swait(NCHUNK - 2, B)
        compute(NCHUNK - 1, A)
        swait(NCHUNK - 1, A)
        plsc.subcore_barrier()

        pltpu.sync_copy(agg_sh.at[pl.ds(sid * RPT, RPT)],
                        agg_out.at[cid, hh, pl.ds(sid * RPT, RPT)])
        if hh == 0:
            pltpu.sync_copy(s_sh.at[pl.ds(sid * RPT, RPT)],
                            s_out.at[cid, pl.ds(sid * RPT, RPT)])

        @pl.when(sid == 0)
        def _dump_rem():
            pltpu.sync_copy(agg_sh.at[pl.ds(NS * RPT, REM)],
                            agg_out.at[cid, hh, pl.ds(NS * RPT, REM)])
            if hh == 0:
                pltpu.sync_copy(s_sh.at[pl.ds(NS * RPT, REM)],
                                s_out.at[cid, pl.ds(NS * RPT, REM)])

        plsc.subcore_barrier()


def _edge_sc(dst, src, st, smax, v0, v1, v2, v3):
    mesh = plsc.VectorSubcoreMesh(core_axis_name="c", subcore_axis_name="s")
    f = pl.kernel(
        _edge_kernel,
        mesh=mesh,
        out_type=[
            jax.ShapeDtypeStruct((NC, HEADS, N, DIM), f32),
            jax.ShapeDtypeStruct((NC, N, 8), f32),
        ],
        compiler_params=pltpu.CompilerParams(
            needs_layout_passes=False, use_tc_tiling_on_sc=False),
        scratch_types=[
            pltpu.VMEM((NCHUNK, C), jnp.int32),
            pltpu.VMEM((NCHUNK, C), jnp.int32),
            pltpu.VMEM((C, 8), f32),
            pltpu.VMEM((C, 8), f32),
            pltpu.VMEM((C, DIM), f32),
            pltpu.VMEM((C, 8), f32),
            pltpu.VMEM((C, 8), f32),
            pltpu.VMEM((C, DIM), f32),
            pltpu.VMEM((C, 8), f32),
            pltpu.VMEM((C, 8), f32),
            pltpu.VMEM((EPW, 4), f32),
            pltpu.VMEM((16,), f32),
            pltpu.VMEM_SHARED((N, DIM), f32),
            pltpu.VMEM_SHARED((N, 8), f32),
            pltpu.SemaphoreType.DMA,
            pltpu.SemaphoreType.DMA,
            pltpu.SemaphoreType.DMA,
            pltpu.SemaphoreType.DMA,
        ],
    )
    return f(dst, src, st, smax, v0, v1, v2, v3)


def kernel(x, edge_index, K, Wk_w, Wk_b, Wq, Wv, a_param, Wm, b_param, ln_g,
           ln_b, gate_w, gate_b, Win_w, Win_b, Wout_w, Wout_b):
    src = edge_index[0].reshape(NW, NCHUNK, C)
    dst = edge_index[1].reshape(NW, NCHUNK, C)
    kcol = K[:, None]

    h, v0, v1, v2, v3, st, sm, g = _fuse0_call(
        x, kcol, Win_w, Win_b[None, :], Wq[0], Wv[0], Wk_w[0][None, :],
        Wk_b[0][None, :], a_param[0], gate_w[0], gate_b[0][None, :])

    for l in range(NLAYERS):
        agg, s = _edge_sc(dst, src, st, sm.reshape(16), v0, v1, v2, v3)
        if l < NLAYERS - 1:
            h, v0, v1, v2, v3, st, sm, g = _fusemid_call(
                h, agg, s, g, Wm[l], b_param[l][None, :], ln_g[l][None, :],
                ln_b[l][None, :], kcol, Wq[l + 1], Wv[l + 1],
                Wk_w[l + 1][None, :], Wk_b[l + 1][None, :], a_param[l + 1],
                gate_w[l + 1], gate_b[l + 1][None, :])
        else:
            out = _fuselast_call(
                h, agg, s, g, Wm[l], b_param[l][None, :], ln_g[l][None, :],
                ln_b[l][None, :], Wout_w, Wout_b[None, :])
    return out


# speedup vs baseline: 1.0063x; 1.0063x over previous
import jax
import jax.numpy as jnp
from jax import lax
from jax.experimental import pallas as pl
from jax.experimental.pallas import tpu as pltpu, tpu_sc as plsc

N = 10000
E = 160000
IN_DIM = 128
DIM = 64
HEADS = 4
NLAYERS = 3

NC = 2
NS = 16
NW = NC * NS
EPW = E // NW
C = 200
NCHUNK = EPW // C
RPT = 624
REM = N - RPT * NS

f32 = jnp.float32


def _proj_in_body(x_ref, w_ref, b_ref, o_ref):
    o_ref[...] = jax.nn.relu(
        jnp.dot(x_ref[...], w_ref[...].T, preferred_element_type=f32) + b_ref[...])


def _proj_out_body(h_ref, w_ref, b_ref, o_ref):
    o_ref[...] = jnp.dot(h_ref[...], w_ref[...].T, preferred_element_type=f32) + b_ref[...]


def _pre_body(h_ref, k_ref, wq_ref, wv_ref, wkw_ref, wkb_ref, a_ref, gw_ref,
              gb_ref, v0_ref, v1_ref, v2_ref, v3_ref, st_ref, sm_ref, g_ref):
    h = h_ref[...]
    kcol = k_ref[...]
    nb = h.shape[0]
    ku = kcol * wkw_ref[...] + wkb_ref[...]
    q = jnp.dot(h, wq_ref[...].T, preferred_element_type=f32)
    v = jnp.dot(h, wv_ref[...].T, preferred_element_type=f32)
    a = a_ref[...]
    a_k = a[:, :DIM]
    sd = jnp.dot(ku, a_k.T, preferred_element_type=f32)
    sd_parts = []
    ss_parts = []
    for hh in range(HEADS):
        a_q = a[hh:hh + 1, DIM:2 * DIM]
        a_v = a[hh:hh + 1, 2 * DIM:]
        qh = q[:, hh * DIM:(hh + 1) * DIM]
        vh = v[:, hh * DIM:(hh + 1) * DIM]
        sd_parts.append(jnp.sum(qh * a_q, axis=1, keepdims=True))
        ss_parts.append(jnp.sum(vh * a_v, axis=1, keepdims=True))
    sd = sd + jnp.concatenate(sd_parts, axis=1)
    ss = jnp.concatenate(ss_parts, axis=1)
    ssmax = jnp.max(ss, axis=0, keepdims=True)
    st_ref[...] = jnp.concatenate([sd, ss], axis=1)
    cur = jnp.concatenate([ssmax] * 4, axis=1)
    step = pl.program_id(0)

    @pl.when(step == 0)
    def _init_sm():
        sm_ref[...] = cur

    @pl.when(step != 0)
    def _acc_sm():
        sm_ref[...] = jnp.maximum(sm_ref[...], cur)
    v0_ref[...] = v[:, :DIM]
    v1_ref[...] = v[:, DIM:2 * DIM]
    v2_ref[...] = v[:, 2 * DIM:3 * DIM]
    v3_ref[...] = v[:, 3 * DIM:]
    g_ref[...] = jax.nn.sigmoid(
        jnp.dot(ku, gw_ref[...].T, preferred_element_type=f32) + gb_ref[...])


def _post_body(h_ref, agg_ref, s_ref, g_ref, wm_ref, b_ref, lng_ref, lnb_ref,
               o_ref):
    h = h_ref[...]
    s = s_ref[0, :, :HEADS] + s_ref[1, :, :HEADS] + 1e-16
    wm = wm_ref[...]
    out = h
    for hh in range(HEADS):
        agg_h = agg_ref[0, hh] + agg_ref[1, hh]
        agg_h = agg_h / jnp.broadcast_to(s[:, hh:hh + 1], agg_h.shape)
        out = out + jnp.dot(agg_h, wm[:, hh * DIM:(hh + 1) * DIM].T,
                            preferred_element_type=f32)
    mu = jnp.mean(out, axis=-1, keepdims=True)
    var = jnp.mean((out - mu) ** 2, axis=-1, keepdims=True)
    out = (out - mu) * jax.lax.rsqrt(var + 1e-5) * lng_ref[...] + lnb_ref[...]
    out = jax.nn.relu(out * (1.0 + g_ref[...]) + b_ref[...])
    o_ref[...] = out


def _tc_call(body, out_shapes, *args):
    return pl.pallas_call(body, out_shape=out_shapes)(*args)


NB = 2000


def _row_spec(cols):
    return pl.BlockSpec((NB, cols), lambda i: (i, 0))


def _fix_spec(shape):
    nd = len(shape)
    return pl.BlockSpec(shape, lambda i, _n=nd: (0,) * _n)


def _pre_call(h, kcol, wq, wv, wkw, wkb, a, gw, gb):
    return pl.pallas_call(
        _pre_body,
        grid=(N // NB,),
        in_specs=[
            _row_spec(DIM), _row_spec(1),
            _fix_spec((HEADS * DIM, DIM)), _fix_spec((HEADS * DIM, DIM)),
            _fix_spec((1, DIM)), _fix_spec((1, DIM)),
            _fix_spec((HEADS, 3 * DIM)),
            _fix_spec((DIM, DIM)), _fix_spec((1, DIM)),
        ],
        out_specs=[
            _row_spec(DIM), _row_spec(DIM), _row_spec(DIM), _row_spec(DIM),
            _row_spec(8), _fix_spec((1, 16)), _row_spec(DIM),
        ],
        out_shape=[
            jax.ShapeDtypeStruct((N, DIM), f32),
            jax.ShapeDtypeStruct((N, DIM), f32),
            jax.ShapeDtypeStruct((N, DIM), f32),
            jax.ShapeDtypeStruct((N, DIM), f32),
            jax.ShapeDtypeStruct((N, 8), f32),
            jax.ShapeDtypeStruct((1, 16), f32),
            jax.ShapeDtypeStruct((N, DIM), f32),
        ],
    )(h, kcol, wq, wv, wkw, wkb, a, gw, gb)


def _post_call(h, agg, s, g, wm, b, lng, lnb):
    return pl.pallas_call(
        _post_body,
        grid=(N // NB,),
        in_specs=[
            _row_spec(DIM),
            pl.BlockSpec((NC, HEADS, NB, DIM), lambda i: (0, 0, i, 0)),
            pl.BlockSpec((NC, NB, 8), lambda i: (0, i, 0)),
            _row_spec(DIM),
            _fix_spec((DIM, HEADS * DIM)),
            _fix_spec((1, DIM)), _fix_spec((1, DIM)), _fix_spec((1, DIM)),
        ],
        out_specs=_row_spec(DIM),
        out_shape=jax.ShapeDtypeStruct((N, DIM), f32),
    )(h, agg, s, g, wm, b, lng, lnb)


def _pre_compute(h, kcol, wq, wv, wkw, wkb, a, gw, gb,
                 v0_ref, v1_ref, v2_ref, v3_ref, st_ref, sm_ref, g_ref):
    ku = kcol * wkw + wkb
    q = jnp.dot(h, wq.T, preferred_element_type=f32)
    v = jnp.dot(h, wv.T, preferred_element_type=f32)
    a_k = a[:, :DIM]
    sd = jnp.dot(ku, a_k.T, preferred_element_type=f32)
    sd_parts = []
    ss_parts = []
    for hh in range(HEADS):
        a_q = a[hh:hh + 1, DIM:2 * DIM]
        a_v = a[hh:hh + 1, 2 * DIM:]
        qh = q[:, hh * DIM:(hh + 1) * DIM]
        vh = v[:, hh * DIM:(hh + 1) * DIM]
        sd_parts.append(jnp.sum(qh * a_q, axis=1, keepdims=True))
        ss_parts.append(jnp.sum(vh * a_v, axis=1, keepdims=True))
    sd = sd + jnp.concatenate(sd_parts, axis=1)
    ss = jnp.concatenate(ss_parts, axis=1)
    ssmax = jnp.max(ss, axis=0, keepdims=True)
    st_ref[...] = jnp.concatenate([sd, ss], axis=1)
    cur = jnp.concatenate([ssmax] * 4, axis=1)
    step = pl.program_id(0)

    @pl.when(step == 0)
    def _init_sm():
        sm_ref[...] = cur

    @pl.when(step != 0)
    def _acc_sm():
        sm_ref[...] = jnp.maximum(sm_ref[...], cur)
    v0_ref[...] = v[:, :DIM]
    v1_ref[...] = v[:, DIM:2 * DIM]
    v2_ref[...] = v[:, 2 * DIM:3 * DIM]
    v3_ref[...] = v[:, 3 * DIM:]
    g_ref[...] = jax.nn.sigmoid(
        jnp.dot(ku, gw.T, preferred_element_type=f32) + gb)


def _post_compute(h, agg_ref, s_ref, g, wm, b, lng, lnb):
    s = s_ref[0, :, :HEADS] + s_ref[1, :, :HEADS] + 1e-16
    out = h
    for hh in range(HEADS):
        agg_h = agg_ref[0, hh] + agg_ref[1, hh]
        agg_h = agg_h / jnp.broadcast_to(s[:, hh:hh + 1], agg_h.shape)
        out = out + jnp.dot(agg_h, wm[:, hh * DIM:(hh + 1) * DIM].T,
                            preferred_element_type=f32)
    mu = jnp.mean(out, axis=-1, keepdims=True)
    var = jnp.mean((out - mu) ** 2, axis=-1, keepdims=True)
    out = (out - mu) * jax.lax.rsqrt(var + 1e-5) * lng + lnb
    return jax.nn.relu(out * (1.0 + g) + b)


def _fuse0_body(x_ref, k_ref, winw_ref, winb_ref, wq_ref, wv_ref, wkw_ref,
                wkb_ref, a_ref, gw_ref, gb_ref,
                h_ref, v0_ref, v1_ref, v2_ref, v3_ref, st_ref, sm_ref, g_ref):
    h = jax.nn.relu(jnp.dot(x_ref[...], winw_ref[...].T,
                            preferred_element_type=f32) + winb_ref[...])
    h_ref[...] = h
    _pre_compute(h, k_ref[...], wq_ref[...], wv_ref[...], wkw_ref[...],
                 wkb_ref[...], a_ref[...], gw_ref[...], gb_ref[...],
                 v0_ref, v1_ref, v2_ref, v3_ref, st_ref, sm_ref, g_ref)


def _fusemid_body(h_ref, agg_ref, s_ref, gin_ref, wm_ref, b_ref, lng_ref,
                  lnb_ref, k_ref, wq_ref, wv_ref, wkw_ref, wkb_ref, a_ref,
                  gw_ref, gb_ref,
                  h_out, v0_ref, v1_ref, v2_ref, v3_ref, st_ref, sm_ref, g_ref):
    h2 = _post_compute(h_ref[...], agg_ref, s_ref, gin_ref[...], wm_ref[...],
                       b_ref[...], lng_ref[...], lnb_ref[...])
    h_out[...] = h2
    _pre_compute(h2, k_ref[...], wq_ref[...], wv_ref[...], wkw_ref[...],
                 wkb_ref[...], a_ref[...], gw_ref[...], gb_ref[...],
                 v0_ref, v1_ref, v2_ref, v3_ref, st_ref, sm_ref, g_ref)


def _fuselast_body(h_ref, agg_ref, s_ref, gin_ref, wm_ref, b_ref, lng_ref,
                   lnb_ref, woutw_ref, woutb_ref, o_ref):
    h2 = _post_compute(h_ref[...], agg_ref, s_ref, gin_ref[...], wm_ref[...],
                       b_ref[...], lng_ref[...], lnb_ref[...])
    o_ref[...] = jnp.dot(h2, woutw_ref[...].T,
                         preferred_element_type=f32) + woutb_ref[...]


_PRE_W_SPECS = [
    _fix_spec((HEADS * DIM, DIM)), _fix_spec((HEADS * DIM, DIM)),
    _fix_spec((1, DIM)), _fix_spec((1, DIM)), _fix_spec((HEADS, 3 * DIM)),
    _fix_spec((DIM, DIM)), _fix_spec((1, DIM)),
]
_PRE_OUT_SPECS = [
    _row_spec(DIM), _row_spec(DIM), _row_spec(DIM), _row_spec(DIM),
    _row_spec(8), _fix_spec((1, 16)), _row_spec(DIM),
]
_PRE_OUT_SHAPES = [
    jax.ShapeDtypeStruct((N, DIM), f32),
    jax.ShapeDtypeStruct((N, DIM), f32),
    jax.ShapeDtypeStruct((N, DIM), f32),
    jax.ShapeDtypeStruct((N, DIM), f32),
    jax.ShapeDtypeStruct((N, 8), f32),
    jax.ShapeDtypeStruct((1, 16), f32),
    jax.ShapeDtypeStruct((N, DIM), f32),
]
_POST_IN_SPECS = [
    _row_spec(DIM),
    pl.BlockSpec((NC, HEADS, NB, DIM), lambda i: (0, 0, i, 0)),
    pl.BlockSpec((NC, NB, 8), lambda i: (0, i, 0)),
    _row_spec(DIM),
    _fix_spec((DIM, HEADS * DIM)),
    _fix_spec((1, DIM)), _fix_spec((1, DIM)), _fix_spec((1, DIM)),
]


def _fuse0_call(x, kcol, winw, winb, wq, wv, wkw, wkb, a, gw, gb):
    return pl.pallas_call(
        _fuse0_body,
        grid=(N // NB,),
        in_specs=[_row_spec(IN_DIM), _row_spec(1),
                  _fix_spec((DIM, IN_DIM)), _fix_spec((1, DIM))] + _PRE_W_SPECS,
        out_specs=[_row_spec(DIM)] + _PRE_OUT_SPECS,
        out_shape=[jax.ShapeDtypeStruct((N, DIM), f32)] + _PRE_OUT_SHAPES,
    )(x, kcol, winw, winb, wq, wv, wkw, wkb, a, gw, gb)


def _fusemid_call(h, agg, s, g, wm, b, lng, lnb, kcol, wq, wv, wkw, wkb, a,
                  gw, gb):
    return pl.pallas_call(
        _fusemid_body,
        grid=(N // NB,),
        in_specs=_POST_IN_SPECS + [_row_spec(1)] + _PRE_W_SPECS,
        out_specs=[_row_spec(DIM)] + _PRE_OUT_SPECS,
        out_shape=[jax.ShapeDtypeStruct((N, DIM), f32)] + _PRE_OUT_SHAPES,
    )(h, agg, s, g, wm, b, lng, lnb, kcol, wq, wv, wkw, wkb, a, gw, gb)


def _fuselast_call(h, agg, s, g, wm, b, lng, lnb, woutw, woutb):
    return pl.pallas_call(
        _fuselast_body,
        grid=(N // NB,),
        in_specs=_POST_IN_SPECS + [_fix_spec((DIM, DIM)), _fix_spec((1, DIM))],
        out_specs=_row_spec(DIM),
        out_shape=jax.ShapeDtypeStruct((N, DIM), f32),
    )(h, agg, s, g, wm, b, lng, lnb, woutw, woutb)


def _edge_kernel(dst_hbm, src_hbm, st_hbm, smax_hbm, v0, v1, v2, v3,
                 agg_out, s_out,
                 dst2, src2, drows_a, srows_a, vbuf_a, drows_b, srows_b,
                 vbuf_b, w4, w4all, smaxv, agg_sh, s_sh,
                 gsem_a, gsem_b, ssem_a, ssem_b):
    cid = lax.axis_index("c")
    sid = lax.axis_index("s")
    wid = sid * NC + cid

    iota = lax.iota(jnp.int32, 16)
    lane4 = iota % 4
    edge4 = iota // 4
    zeros16 = jnp.zeros((16,), f32)

    def _full(val):
        return jnp.full((16,), val, jnp.int32)

    pltpu.sync_copy(smax_hbm, smaxv)
    smax16 = smaxv[...]

    pltpu.sync_copy(dst_hbm.at[wid], dst2)
    pltpu.sync_copy(src_hbm.at[wid], src2)

    def _zv(i, _):
        plsc.store_scatter(vbuf_a, [_full(i // 4), (i % 4) * 16 + iota], zeros16)
        return _

    def _z3(i, _):
        p = i * 16 + iota
        plsc.store_scatter(w4, [p // 8, p % 8], zeros16)
        return _
    lax.fori_loop(0, C // 2, _z3, None)

    for hh in range(HEADS):
        v_hbm = (v0, v1, v2, v3)[hh]
        lax.fori_loop(0, C * DIM // 16, _zv, None)
        for zc in range(3):
            pltpu.sync_copy(vbuf_a, agg_sh.at[pl.ds(sid * RPT + zc * C, C)])
        pltpu.sync_copy(vbuf_a.at[pl.ds(0, 24)],
                        agg_sh.at[pl.ds(sid * RPT + 3 * C, 24)])
        if hh == 0:
            for zc in range(3):
                pltpu.sync_copy(w4, s_sh.at[pl.ds(sid * RPT + zc * C, C)])
            pltpu.sync_copy(w4.at[pl.ds(0, 24)],
                            s_sh.at[pl.ds(sid * RPT + 3 * C, 24)])

        @pl.when(sid == 0)
        def _zero_rem():
            pltpu.sync_copy(vbuf_a.at[pl.ds(0, REM)],
                            agg_sh.at[pl.ds(NS * RPT, REM)])
            if hh == 0:
                pltpu.sync_copy(w4.at[pl.ds(0, REM)],
                                s_sh.at[pl.ds(NS * RPT, REM)])

        plsc.subcore_barrier()

        A = (drows_a, srows_a, vbuf_a, gsem_a, ssem_a)
        B = (drows_b, srows_b, vbuf_b, gsem_b, ssem_b)

        def gstart(kk, S):
            dr, sr, vb, gsem, _ = S
            if hh == 0:
                pltpu.async_copy(st_hbm.at[dst2.at[kk]], dr, gsem)
                pltpu.async_copy(st_hbm.at[src2.at[kk]], sr, gsem)
            pltpu.async_copy(v_hbm.at[src2.at[kk]], vb, gsem)

        def gwait(kk, S):
            dr, sr, vb, gsem, _ = S
            if hh == 0:
                pltpu.make_async_copy(st_hbm.at[dst2.at[kk]], dr, gsem).wait()
                pltpu.make_async_copy(st_hbm.at[src2.at[kk]], sr, gsem).wait()
            pltpu.make_async_copy(v_hbm.at[src2.at[kk]], vb, gsem).wait()

        def compute(kk, S):
            dr, sr, vb, _, ssem = S

            if hh == 0:
                def wstep(g, _):
                    r = g * 4 + edge4
                    sdv = plsc.load_gather(dr, [r, lane4])
                    ssv = plsc.load_gather(sr, [r, lane4 + 4])
                    tb = sdv + smax16
                    ubv = jnp.maximum(tb, 0.01 * tb)
                    t = sdv + ssv
                    t = jnp.maximum(t, 0.01 * t)
                    w = jnp.exp(t - ubv)
                    plsc.store_scatter(w4, [r, lane4], w)
                    plsc.store_scatter(w4all, [kk * C + r, lane4], w)
                    return _
                lax.fori_loop(0, C // 4, wstep, None, unroll=4)

            def mstep(e, _):
                wv = plsc.load_gather(w4all, [_full(kk * C + e), _full(hh)])
                for j in range(DIM // 16):
                    xv = vb[e, pl.ds(j * 16, 16)]
                    vb[e, pl.ds(j * 16, 16)] = xv * wv
                return _
            lax.fori_loop(0, C, mstep, None, unroll=4)

            pltpu.async_copy(vb, agg_sh.at[dst2.at[kk]], ssem, add=True)
            if hh == 0:
                pltpu.sync_copy(w4, s_sh.at[dst2.at[kk]], add=True)

        def swait(kk, S):
            _, _, vb, _, ssem = S
            pltpu.make_async_copy(vb, agg_sh.at[dst2.at[kk]], ssem).wait()

        gstart(0, A)

        def pair(kp, _):
            kk = 2 * kp
            gwait(kk, A)

            @pl.when(kp > 0)
            def _wb():
                swait(kk - 1, B)

            gstart(kk + 1, B)
            compute(kk, A)
            gwait(kk + 1, B)
            swait(kk, A)
            gstart(kk + 2, A)
            compute(kk + 1, B)
            return _
        lax.fori_loop(0, (NCHUNK - 1) // 2, pair, None)

        gwait(NCHUNK - 1, A)
        swait(NCHUNK - 2, B)
        compute(NCHUNK - 1, A)
        swait(NCHUNK - 1, A)
        plsc.subcore_barrier()

        pltpu.sync_copy(agg_sh.at[pl.ds(sid * RPT, RPT)],
                        agg_out.at[cid, hh, pl.ds(sid * RPT, RPT)])
        if hh == 0:
            pltpu.sync_copy(s_sh.at[pl.ds(sid * RPT, RPT)],
                            s_out.at[cid, pl.ds(sid * RPT, RPT)])

        @pl.when(sid == 0)
        def _dump_rem():
            pltpu.sync_copy(agg_sh.at[pl.ds(NS * RPT, REM)],
                            agg_out.at[cid, hh, pl.ds(NS * RPT, REM)])
            if hh == 0:
                pltpu.sync_copy(s_sh.at[pl.ds(NS * RPT, REM)],
                                s_out.at[cid, pl.ds(NS * RPT, REM)])

        plsc.subcore_barrier()


def _edge_sc(dst, src, st, smax, v0, v1, v2, v3):
    mesh = plsc.VectorSubcoreMesh(core_axis_name="c", subcore_axis_name="s")
    f = pl.kernel(
        _edge_kernel,
        mesh=mesh,
        out_type=[
            jax.ShapeDtypeStruct((NC, HEADS, N, DIM), f32),
            jax.ShapeDtypeStruct((NC, N, 8), f32),
        ],
        compiler_params=pltpu.CompilerParams(
            needs_layout_passes=False, use_tc_tiling_on_sc=False),
        scratch_types=[
            pltpu.VMEM((NCHUNK, C), jnp.int32),
            pltpu.VMEM((NCHUNK, C), jnp.int32),
            pltpu.VMEM((C, 8), f32),
            pltpu.VMEM((C, 8), f32),
            pltpu.VMEM((C, DIM), f32),
            pltpu.VMEM((C, 8), f32),
            pltpu.VMEM((C, 8), f32),
            pltpu.VMEM((C, DIM), f32),
            pltpu.VMEM((C, 8), f32),
            pltpu.VMEM((EPW, 4), f32),
            pltpu.VMEM((16,), f32),
            pltpu.VMEM_SHARED((N, DIM), f32),
            pltpu.VMEM_SHARED((N, 8), f32),
            pltpu.SemaphoreType.DMA,
            pltpu.SemaphoreType.DMA,
            pltpu.SemaphoreType.DMA,
            pltpu.SemaphoreType.DMA,
        ],
    )
    return f(dst, src, st, smax, v0, v1, v2, v3)


def kernel(x, edge_index, K, Wk_w, Wk_b, Wq, Wv, a_param, Wm, b_param, ln_g,
           ln_b, gate_w, gate_b, Win_w, Win_b, Wout_w, Wout_b):
    src = edge_index[0].reshape(NW, NCHUNK, C)
    dst = edge_index[1].reshape(NW, NCHUNK, C)
    kcol = K[:, None]

    h, v0, v1, v2, v3, st, sm, g = _fuse0_call(
        x, kcol, Win_w, Win_b[None, :], Wq[0], Wv[0], Wk_w[0][None, :],
        Wk_b[0][None, :], a_param[0], gate_w[0], gate_b[0][None, :])

    for l in range(NLAYERS):
        agg, s = _edge_sc(dst, src, st, sm.reshape(16), v0, v1, v2, v3)
        if l < NLAYERS - 1:
            h, v0, v1, v2, v3, st, sm, g = _fusemid_call(
                h, agg, s, g, Wm[l], b_param[l][None, :], ln_g[l][None, :],
                ln_b[l][None, :], kcol, Wq[l + 1], Wv[l + 1],
                Wk_w[l + 1][None, :], Wk_b[l + 1][None, :], a_param[l + 1],
                gate_w[l + 1], gate_b[l + 1][None, :])
        else:
            out = _fuselast_call(
                h, agg, s, g, Wm[l], b_param[l][None, :], ln_g[l][None, :],
                ln_b[l][None, :], Wout_w, Wout_b[None, :])
    return out


# speedup vs baseline: 1.0064x; 1.0001x over previous
import jax
import jax.numpy as jnp
from jax import lax
from jax.experimental import pallas as pl
from jax.experimental.pallas import tpu as pltpu, tpu_sc as plsc

N = 10000
E = 160000
IN_DIM = 128
DIM = 64
HEADS = 4
NLAYERS = 3

NC = 2
NS = 16
NW = NC * NS
EPW = E // NW
C = 200
NCHUNK = EPW // C
RPT = 624
REM = N - RPT * NS

f32 = jnp.float32


NB = 2000


def _row_spec(cols):
    return pl.BlockSpec((NB, cols), lambda i: (i, 0))


def _fix_spec(shape):
    nd = len(shape)
    return pl.BlockSpec(shape, lambda i, _n=nd: (0,) * _n)


def _pre_compute(h, kcol, wq, wv, wkw, wkb, a, gw, gb,
                 v0_ref, v1_ref, v2_ref, v3_ref, st_ref, sm_ref, g_ref):
    ku = kcol * wkw + wkb
    q = jnp.dot(h, wq.T, preferred_element_type=f32)
    v = jnp.dot(h, wv.T, preferred_element_type=f32)
    a_k = a[:, :DIM]
    sd = jnp.dot(ku, a_k.T, preferred_element_type=f32)
    sd_parts = []
    ss_parts = []
    for hh in range(HEADS):
        a_q = a[hh:hh + 1, DIM:2 * DIM]
        a_v = a[hh:hh + 1, 2 * DIM:]
        qh = q[:, hh * DIM:(hh + 1) * DIM]
        vh = v[:, hh * DIM:(hh + 1) * DIM]
        sd_parts.append(jnp.sum(qh * a_q, axis=1, keepdims=True))
        ss_parts.append(jnp.sum(vh * a_v, axis=1, keepdims=True))
    sd = sd + jnp.concatenate(sd_parts, axis=1)
    ss = jnp.concatenate(ss_parts, axis=1)
    ssmax = jnp.max(ss, axis=0, keepdims=True)
    st_ref[...] = jnp.concatenate([sd, ss], axis=1)
    cur = jnp.concatenate([ssmax] * 4, axis=1)
    step = pl.program_id(0)

    @pl.when(step == 0)
    def _init_sm():
        sm_ref[...] = cur

    @pl.when(step != 0)
    def _acc_sm():
        sm_ref[...] = jnp.maximum(sm_ref[...], cur)
    v0_ref[...] = v[:, :DIM]
    v1_ref[...] = v[:, DIM:2 * DIM]
    v2_ref[...] = v[:, 2 * DIM:3 * DIM]
    v3_ref[...] = v[:, 3 * DIM:]
    g_ref[...] = jax.nn.sigmoid(
        jnp.dot(ku, gw.T, preferred_element_type=f32) + gb)


def _post_compute(h, agg_ref, s_ref, g, wm, b, lng, lnb):
    s = s_ref[0, :, :HEADS] + s_ref[1, :, :HEADS] + 1e-16
    out = h
    for hh in range(HEADS):
        agg_h = agg_ref[0, hh] + agg_ref[1, hh]
        agg_h = agg_h / jnp.broadcast_to(s[:, hh:hh + 1], agg_h.shape)
        out = out + jnp.dot(agg_h, wm[:, hh * DIM:(hh + 1) * DIM].T,
                            preferred_element_type=f32)
    mu = jnp.mean(out, axis=-1, keepdims=True)
    var = jnp.mean((out - mu) ** 2, axis=-1, keepdims=True)
    out = (out - mu) * jax.lax.rsqrt(var + 1e-5) * lng + lnb
    return jax.nn.relu(out * (1.0 + g) + b)


def _fuse0_body(x_ref, k_ref, winw_ref, winb_ref, wq_ref, wv_ref, wkw_ref,
                wkb_ref, a_ref, gw_ref, gb_ref,
                h_ref, v0_ref, v1_ref, v2_ref, v3_ref, st_ref, sm_ref, g_ref):
    h = jax.nn.relu(jnp.dot(x_ref[...], winw_ref[...].T,
                            preferred_element_type=f32) + winb_ref[...])
    h_ref[...] = h
    _pre_compute(h, k_ref[...], wq_ref[...], wv_ref[...], wkw_ref[...],
                 wkb_ref[...], a_ref[...], gw_ref[...], gb_ref[...],
                 v0_ref, v1_ref, v2_ref, v3_ref, st_ref, sm_ref, g_ref)


def _fusemid_body(h_ref, agg_ref, s_ref, gin_ref, wm_ref, b_ref, lng_ref,
                  lnb_ref, k_ref, wq_ref, wv_ref, wkw_ref, wkb_ref, a_ref,
                  gw_ref, gb_ref,
                  h_out, v0_ref, v1_ref, v2_ref, v3_ref, st_ref, sm_ref, g_ref):
    h2 = _post_compute(h_ref[...], agg_ref, s_ref, gin_ref[...], wm_ref[...],
                       b_ref[...], lng_ref[...], lnb_ref[...])
    h_out[...] = h2
    _pre_compute(h2, k_ref[...], wq_ref[...], wv_ref[...], wkw_ref[...],
                 wkb_ref[...], a_ref[...], gw_ref[...], gb_ref[...],
                 v0_ref, v1_ref, v2_ref, v3_ref, st_ref, sm_ref, g_ref)


def _fuselast_body(h_ref, agg_ref, s_ref, gin_ref, wm_ref, b_ref, lng_ref,
                   lnb_ref, woutw_ref, woutb_ref, o_ref):
    h2 = _post_compute(h_ref[...], agg_ref, s_ref, gin_ref[...], wm_ref[...],
                       b_ref[...], lng_ref[...], lnb_ref[...])
    o_ref[...] = jnp.dot(h2, woutw_ref[...].T,
                         preferred_element_type=f32) + woutb_ref[...]


_PRE_W_SPECS = [
    _fix_spec((HEADS * DIM, DIM)), _fix_spec((HEADS * DIM, DIM)),
    _fix_spec((1, DIM)), _fix_spec((1, DIM)), _fix_spec((HEADS, 3 * DIM)),
    _fix_spec((DIM, DIM)), _fix_spec((1, DIM)),
]
_PRE_OUT_SPECS = [
    _row_spec(DIM), _row_spec(DIM), _row_spec(DIM), _row_spec(DIM),
    _row_spec(8), _fix_spec((1, 16)), _row_spec(DIM),
]
_PRE_OUT_SHAPES = [
    jax.ShapeDtypeStruct((N, DIM), f32),
    jax.ShapeDtypeStruct((N, DIM), f32),
    jax.ShapeDtypeStruct((N, DIM), f32),
    jax.ShapeDtypeStruct((N, DIM), f32),
    jax.ShapeDtypeStruct((N, 8), f32),
    jax.ShapeDtypeStruct((1, 16), f32),
    jax.ShapeDtypeStruct((N, DIM), f32),
]
_POST_IN_SPECS = [
    _row_spec(DIM),
    pl.BlockSpec((NC, HEADS, NB, DIM), lambda i: (0, 0, i, 0)),
    pl.BlockSpec((NC, NB, 8), lambda i: (0, i, 0)),
    _row_spec(DIM),
    _fix_spec((DIM, HEADS * DIM)),
    _fix_spec((1, DIM)), _fix_spec((1, DIM)), _fix_spec((1, DIM)),
]


def _fuse0_call(x, kcol, winw, winb, wq, wv, wkw, wkb, a, gw, gb):
    return pl.pallas_call(
        _fuse0_body,
        grid=(N // NB,),
        in_specs=[_row_spec(IN_DIM), _row_spec(1),
                  _fix_spec((DIM, IN_DIM)), _fix_spec((1, DIM))] + _PRE_W_SPECS,
        out_specs=[_row_spec(DIM)] + _PRE_OUT_SPECS,
        out_shape=[jax.ShapeDtypeStruct((N, DIM), f32)] + _PRE_OUT_SHAPES,
    )(x, kcol, winw, winb, wq, wv, wkw, wkb, a, gw, gb)


def _fusemid_call(h, agg, s, g, wm, b, lng, lnb, kcol, wq, wv, wkw, wkb, a,
                  gw, gb):
    return pl.pallas_call(
        _fusemid_body,
        grid=(N // NB,),
        in_specs=_POST_IN_SPECS + [_row_spec(1)] + _PRE_W_SPECS,
        out_specs=[_row_spec(DIM)] + _PRE_OUT_SPECS,
        out_shape=[jax.ShapeDtypeStruct((N, DIM), f32)] + _PRE_OUT_SHAPES,
    )(h, agg, s, g, wm, b, lng, lnb, kcol, wq, wv, wkw, wkb, a, gw, gb)


def _fuselast_call(h, agg, s, g, wm, b, lng, lnb, woutw, woutb):
    return pl.pallas_call(
        _fuselast_body,
        grid=(N // NB,),
        in_specs=_POST_IN_SPECS + [_fix_spec((DIM, DIM)), _fix_spec((1, DIM))],
        out_specs=_row_spec(DIM),
        out_shape=jax.ShapeDtypeStruct((N, DIM), f32),
    )(h, agg, s, g, wm, b, lng, lnb, woutw, woutb)


def _edge_kernel(dst_hbm, src_hbm, st_hbm, smax_hbm, v0, v1, v2, v3,
                 agg_out, s_out,
                 dst2, src2, drows_a, srows_a, vbuf_a, drows_b, srows_b,
                 vbuf_b, w4, w4all, smaxv, agg_sh, s_sh,
                 gsem_a, gsem_b, ssem_a, ssem_b):
    cid = lax.axis_index("c")
    sid = lax.axis_index("s")
    wid = sid * NC + cid

    iota = lax.iota(jnp.int32, 16)
    lane4 = iota % 4
    edge4 = iota // 4
    zeros16 = jnp.zeros((16,), f32)

    def _full(val):
        return jnp.full((16,), val, jnp.int32)

    pltpu.sync_copy(smax_hbm, smaxv)
    smax16 = smaxv[...]

    pltpu.sync_copy(dst_hbm.at[wid], dst2)
    pltpu.sync_copy(src_hbm.at[wid], src2)

    def _zv(i, _):
        plsc.store_scatter(vbuf_a, [_full(i // 4), (i % 4) * 16 + iota], zeros16)
        return _

    def _z3(i, _):
        p = i * 16 + iota
        plsc.store_scatter(w4, [p // 8, p % 8], zeros16)
        return _
    lax.fori_loop(0, C // 2, _z3, None)

    for hh in range(HEADS):
        v_hbm = (v0, v1, v2, v3)[hh]
        lax.fori_loop(0, C * DIM // 16, _zv, None)
        for zc in range(3):
            pltpu.sync_copy(vbuf_a, agg_sh.at[pl.ds(sid * RPT + zc * C, C)])
        pltpu.sync_copy(vbuf_a.at[pl.ds(0, 24)],
                        agg_sh.at[pl.ds(sid * RPT + 3 * C, 24)])
        if hh == 0:
            for zc in range(3):
                pltpu.sync_copy(w4, s_sh.at[pl.ds(sid * RPT + zc * C, C)])
            pltpu.sync_copy(w4.at[pl.ds(0, 24)],
                            s_sh.at[pl.ds(sid * RPT + 3 * C, 24)])

        @pl.when(sid == 0)
        def _zero_rem():
            pltpu.sync_copy(vbuf_a.at[pl.ds(0, REM)],
                            agg_sh.at[pl.ds(NS * RPT, REM)])
            if hh == 0:
                pltpu.sync_copy(w4.at[pl.ds(0, REM)],
                                s_sh.at[pl.ds(NS * RPT, REM)])

        plsc.subcore_barrier()

        A = (drows_a, srows_a, vbuf_a, gsem_a, ssem_a)
        B = (drows_b, srows_b, vbuf_b, gsem_b, ssem_b)

        def gstart(kk, S):
            dr, sr, vb, gsem, _ = S
            if hh == 0:
                pltpu.async_copy(st_hbm.at[dst2.at[kk]], dr, gsem)
                pltpu.async_copy(st_hbm.at[src2.at[kk]], sr, gsem)
            pltpu.async_copy(v_hbm.at[src2.at[kk]], vb, gsem)

        def gwait(kk, S):
            dr, sr, vb, gsem, _ = S
            if hh == 0:
                pltpu.make_async_copy(st_hbm.at[dst2.at[kk]], dr, gsem).wait()
                pltpu.make_async_copy(st_hbm.at[src2.at[kk]], sr, gsem).wait()
            pltpu.make_async_copy(v_hbm.at[src2.at[kk]], vb, gsem).wait()

        def compute(kk, S):
            dr, sr, vb, _, ssem = S

            if hh == 0:
                def wstep(g, _):
                    r = g * 4 + edge4
                    sdv = plsc.load_gather(dr, [r, lane4])
                    ssv = plsc.load_gather(sr, [r, lane4 + 4])
                    tb = sdv + smax16
                    ubv = jnp.maximum(tb, 0.01 * tb)
                    t = sdv + ssv
                    t = jnp.maximum(t, 0.01 * t)
                    w = jnp.exp(t - ubv)
                    plsc.store_scatter(w4, [r, lane4], w)
                    plsc.store_scatter(w4all, [kk * C + r, lane4], w)
                    return _
                lax.fori_loop(0, C // 4, wstep, None, unroll=4)

            def mstep(e, _):
                wv = plsc.load_gather(w4all, [_full(kk * C + e), _full(hh)])
                for j in range(DIM // 16):
                    xv = vb[e, pl.ds(j * 16, 16)]
                    vb[e, pl.ds(j * 16, 16)] = xv * wv
                return _
            lax.fori_loop(0, C, mstep, None, unroll=4)

            pltpu.async_copy(vb, agg_sh.at[dst2.at[kk]], ssem, add=True)
            if hh == 0:
                pltpu.sync_copy(w4, s_sh.at[dst2.at[kk]], add=True)

        def swait(kk, S):
            _, _, vb, _, ssem = S
            pltpu.make_async_copy(vb, agg_sh.at[dst2.at[kk]], ssem).wait()

        gstart(0, A)

        def pair(kp, _):
            kk = 2 * kp
            gwait(kk, A)

            @pl.when(kp > 0)
            def _wb():
                swait(kk - 1, B)

            gstart(kk + 1, B)
            compute(kk, A)
            gwait(kk + 1, B)
            swait(kk, A)
            gstart(kk + 2, A)
            compute(kk + 1, B)
            return _
        lax.fori_loop(0, (NCHUNK - 1) // 2, pair, None)

        gwait(NCHUNK - 1, A)
        swait(NCHUNK - 2, B)
        compute(NCHUNK - 1, A)
        swait(NCHUNK - 1, A)
        plsc.subcore_barrier()

        pltpu.sync_copy(agg_sh.at[pl.ds(sid * RPT, RPT)],
                        agg_out.at[cid, hh, pl.ds(sid * RPT, RPT)])
        if hh == 0:
            pltpu.sync_copy(s_sh.at[pl.ds(sid * RPT, RPT)],
                            s_out.at[cid, pl.ds(sid * RPT, RPT)])

        @pl.when(sid == 0)
        def _dump_rem():
            pltpu.sync_copy(agg_sh.at[pl.ds(NS * RPT, REM)],
                            agg_out.at[cid, hh, pl.ds(NS * RPT, REM)])
            if hh == 0:
                pltpu.sync_copy(s_sh.at[pl.ds(NS * RPT, REM)],
                                s_out.at[cid, pl.ds(NS * RPT, REM)])

        plsc.subcore_barrier()


def _edge_sc(dst, src, st, smax, v0, v1, v2, v3):
    mesh = plsc.VectorSubcoreMesh(core_axis_name="c", subcore_axis_name="s")
    f = pl.kernel(
        _edge_kernel,
        mesh=mesh,
        out_type=[
            jax.ShapeDtypeStruct((NC, HEADS, N, DIM), f32),
            jax.ShapeDtypeStruct((NC, N, 8), f32),
        ],
        compiler_params=pltpu.CompilerParams(
            needs_layout_passes=False, use_tc_tiling_on_sc=False),
        scratch_types=[
            pltpu.VMEM((NCHUNK, C), jnp.int32),
            pltpu.VMEM((NCHUNK, C), jnp.int32),
            pltpu.VMEM((C, 8), f32),
            pltpu.VMEM((C, 8), f32),
            pltpu.VMEM((C, DIM), f32),
            pltpu.VMEM((C, 8), f32),
            pltpu.VMEM((C, 8), f32),
            pltpu.VMEM((C, DIM), f32),
            pltpu.VMEM((C, 8), f32),
            pltpu.VMEM((EPW, 4), f32),
            pltpu.VMEM((16,), f32),
            pltpu.VMEM_SHARED((N, DIM), f32),
            pltpu.VMEM_SHARED((N, 8), f32),
            pltpu.SemaphoreType.DMA,
            pltpu.SemaphoreType.DMA,
            pltpu.SemaphoreType.DMA,
            pltpu.SemaphoreType.DMA,
        ],
    )
    return f(dst, src, st, smax, v0, v1, v2, v3)


def kernel(x, edge_index, K, Wk_w, Wk_b, Wq, Wv, a_param, Wm, b_param, ln_g,
           ln_b, gate_w, gate_b, Win_w, Win_b, Wout_w, Wout_b):
    src = edge_index[0].reshape(NW, NCHUNK, C)
    dst = edge_index[1].reshape(NW, NCHUNK, C)
    kcol = K[:, None]

    h, v0, v1, v2, v3, st, sm, g = _fuse0_call(
        x, kcol, Win_w, Win_b[None, :], Wq[0], Wv[0], Wk_w[0][None, :],
        Wk_b[0][None, :], a_param[0], gate_w[0], gate_b[0][None, :])

    for l in range(NLAYERS):
        agg, s = _edge_sc(dst, src, st, sm.reshape(16), v0, v1, v2, v3)
        if l < NLAYERS - 1:
            h, v0, v1, v2, v3, st, sm, g = _fusemid_call(
                h, agg, s, g, Wm[l], b_param[l][None, :], ln_g[l][None, :],
                ln_b[l][None, :], kcol, Wq[l + 1], Wv[l + 1],
                Wk_w[l + 1][None, :], Wk_b[l + 1][None, :], a_param[l + 1],
                gate_w[l + 1], gate_b[l + 1][None, :])
        else:
            out = _fuselast_call(
                h, agg, s, g, Wm[l], b_param[l][None, :], ln_g[l][None, :],
                ln_b[l][None, :], Wout_w, Wout_b[None, :])
    return out
